# baseline (speedup 1.0000x reference)
import jax
import jax.numpy as jnp
from jax import lax
from jax.experimental import pallas as pl
from jax.experimental.pallas import tpu as pltpu
from jax.experimental.pallas import tpu_sc as plsc

N = 10000
D = 128
DH = D // 2
E = 320000
NC = 2
NS = 16
NPAD = 10112
DUMMY = 10008
EC = 128
CHN = 158
EPW = EC * CHN
EPAD = EPW * NS
ROWS_PS = NPAD // NS
NB = 4
GB = 8
BR = NPAD // GB


def _deg_body(edges_hbm, ones_hbm, zeros_hbm, out_hbm, idx_v, ones_v,
              acc_sh, sem):
    c = lax.axis_index("c")
    s = lax.axis_index("s")
    pltpu.sync_copy(zeros_hbm, acc_sh.at[pl.ds(s * ROWS_PS, ROWS_PS)])
    pltpu.sync_copy(edges_hbm.at[c, s], idx_v)
    pltpu.sync_copy(ones_hbm, ones_v)
    plsc.subcore_barrier()
    pend = [None] * 8
    for k in range(CHN):
        b = k % 8
        if pend[b] is not None:
            pend[b].wait()
        pend[b] = pltpu.async_copy(ones_v, acc_sh.at[idx_v.at[k]], sem,
                                   add=True)
    for b in range(8):
        if pend[b] is not None:
            pend[b].wait()
    plsc.subcore_barrier()
    pltpu.sync_copy(acc_sh.at[pl.ds(s * ROWS_PS, ROWS_PS)],
                    out_hbm.at[c, pl.ds(s * ROWS_PS, ROWS_PS)])


def _agg_body(z_hbm, src_hbm, dst_hbm, zeros_hbm, out_hbm, src_v, dst_v,
              bufs, acc_sh, gsems, ssems, zsem):
    c = lax.axis_index("c")
    s = lax.axis_index("s")
    zc = z_hbm.at[c]
    pltpu.sync_copy(src_hbm.at[s], src_v)
    pltpu.sync_copy(dst_hbm.at[s], dst_v)

    gpend = [None] * NB
    spend = [None] * NB
    for k in range(NB - 1):
        gpend[k] = pltpu.async_copy(zc.at[src_v.at[k]], bufs[k], gsems[k])
    zpend = pltpu.async_copy(zeros_hbm,
                             acc_sh.at[pl.ds(s * ROWS_PS, ROWS_PS)], zsem)
    zpend.wait()
    plsc.subcore_barrier()

    for k in range(CHN):
        kn = k + NB - 1
        if kn < CHN:
            bn = kn % NB
            if spend[bn] is not None:
                spend[bn].wait()
            gpend[bn] = pltpu.async_copy(zc.at[src_v.at[kn]], bufs[bn],
                                         gsems[bn])
        b = k % NB
        gpend[b].wait()
        spend[b] = pltpu.async_copy(bufs[b], acc_sh.at[dst_v.at[k]],
                                    ssems[b], add=True)
    for b in range(NB):
        if spend[b] is not None:
            spend[b].wait()
    plsc.subcore_barrier()
    pltpu.sync_copy(acc_sh.at[pl.ds(s * ROWS_PS, ROWS_PS)],
                    out_hbm.at[pl.ds(s * ROWS_PS, ROWS_PS),
                               pl.ds(c * DH, DH)])


_sc_cache = {}


def _sc_kernels():
    if not _sc_cache:
        mesh = plsc.VectorSubcoreMesh(core_axis_name="c",
                                      subcore_axis_name="s",
                                      num_cores=NC, num_subcores=NS)
        params = pltpu.CompilerParams(use_tc_tiling_on_sc=False)
        _sc_cache["deg"] = pl.kernel(
            _deg_body,
            out_type=jax.ShapeDtypeStruct((NC, NPAD, 16), jnp.float32),
            mesh=mesh,
            scratch_types=[
                pltpu.VMEM((CHN, EC), jnp.int32),
                pltpu.VMEM((EC, 16), jnp.float32),
                pltpu.VMEM_SHARED((NPAD, 16), jnp.float32),
                pltpu.SemaphoreType.DMA,
            ],
            compiler_params=params,
        )
        _sc_cache["agg"] = pl.kernel(
            _agg_body,
            out_type=jax.ShapeDtypeStruct((NPAD, D), jnp.float32),
            mesh=mesh,
            scratch_types=[
                pltpu.VMEM((CHN, EC), jnp.int32),
                pltpu.VMEM((CHN, EC), jnp.int32),
                [pltpu.VMEM((EC, DH), jnp.float32) for _ in range(NB)],
                pltpu.VMEM_SHARED((NPAD, DH), jnp.float32),
                [pltpu.SemaphoreType.DMA for _ in range(NB)],
                [pltpu.SemaphoreType.DMA for _ in range(NB)],
                pltpu.SemaphoreType.DMA,
            ],
            compiler_params=params,
        )
    return _sc_cache["deg"], _sc_cache["agg"]


def _norms(deg_ref):
    ns = lax.rsqrt(jnp.maximum(deg_ref[0, :, 0:1], 1.0))
    nd = lax.rsqrt(jnp.maximum(deg_ref[1, :, 0:1], 1.0))
    return ns, nd


def _mm_first_body(f_ref, deg_ref, w_ref, o_ref):
    ns, _ = _norms(deg_ref)
    o_ref[...] = jnp.dot(f_ref[...] * ns, w_ref[...],
                         preferred_element_type=jnp.float32)


def _mm_mid_body(p_ref, deg_ref, b_ref, w_ref, o_ref):
    ns, nd = _norms(deg_ref)
    h = jnp.maximum(p_ref[...] * nd + b_ref[...], 0.0)
    o_ref[...] = jnp.dot(h * ns, w_ref[...],
                         preferred_element_type=jnp.float32)


def _mm_last_body(p_ref, deg_ref, b_ref, o_ref):
    _, nd = _norms(deg_ref)
    o_ref[...] = p_ref[...] * nd + b_ref[...]


_zfull = jax.ShapeDtypeStruct((NPAD, D), jnp.float32)
_row_spec = pl.BlockSpec((BR, D), lambda i: (i, 0))
_deg_spec = pl.BlockSpec((2, BR, 16), lambda i: (0, i, 0))
_w_spec = pl.BlockSpec((D, D), lambda i: (0, 0))
_b_spec = pl.BlockSpec((1, D), lambda i: (0, 0))

_mm_first = pl.pallas_call(
    _mm_first_body, grid=(GB,),
    in_specs=[_row_spec, _deg_spec, _w_spec],
    out_specs=_row_spec, out_shape=_zfull)
_mm_mid = pl.pallas_call(
    _mm_mid_body, grid=(GB,),
    in_specs=[_row_spec, _deg_spec, _b_spec, _w_spec],
    out_specs=_row_spec, out_shape=_zfull)
_mm_last = pl.pallas_call(
    _mm_last_body, grid=(GB,),
    in_specs=[_row_spec, _deg_spec, _b_spec],
    out_specs=_row_spec, out_shape=_zfull)


def kernel(features, edge_index, W1, b1, W2, b2, W3, b3, W4, b4):
    pad = jnp.full((EPAD - E,), DUMMY, jnp.int32)
    srcp = jnp.concatenate([edge_index[0], pad])
    dstp = jnp.concatenate([edge_index[1], pad])
    srcr = srcp.reshape(NS, CHN, EC)
    dstr = dstp.reshape(NS, CHN, EC)
    edges_deg = jnp.stack([srcr, dstr])
    fpad = jnp.concatenate(
        [features, jnp.zeros((NPAD - N, D), jnp.float32)])

    ones16 = jnp.ones((EC, 16), jnp.float32)
    zeros16 = jnp.zeros((ROWS_PS, 16), jnp.float32)
    zerosD = jnp.zeros((ROWS_PS, DH), jnp.float32)

    deg_kernel, agg_kernel = _sc_kernels()
    deg = deg_kernel(edges_deg, ones16, zeros16)
    z = _mm_first(fpad, deg, W1)
    for (W, b) in ((W2, b1), (W3, b2), (W4, b3)):
        zv = jnp.stack([z[:, :DH], z[:, DH:]])
        part = agg_kernel(zv, srcr, dstr, zerosD)
        z = _mm_mid(part, deg, b.reshape(1, D), W)
    zv = jnp.stack([z[:, :DH], z[:, DH:]])
    part = agg_kernel(zv, srcr, dstr, zerosD)
    out = _mm_last(part, deg, b4.reshape(1, D))
    return out[:N]


# speedup vs baseline: 1.0652x; 1.0652x over previous
import jax
import jax.numpy as jnp
from jax import lax
from jax.experimental import pallas as pl
from jax.experimental.pallas import tpu as pltpu
from jax.experimental.pallas import tpu_sc as plsc

N = 10000
D = 128
DH = D // 2
E = 320000
NC = 2
NS = 16
NPAD = 10112
DUMMY = 10008
EC = 128
CHN = 158
EPW = EC * CHN
EPAD = EPW * NS
ROWS_PS = NPAD // NS
NB = 4
GB = 8
BR = NPAD // GB


def _deg_body(edges_hbm, ones_hbm, zeros_hbm, out_hbm, idx_v, ones_v,
              acc_sh, sem):
    c = lax.axis_index("c")
    s = lax.axis_index("s")
    pltpu.sync_copy(zeros_hbm, acc_sh.at[pl.ds(s * ROWS_PS, ROWS_PS)])
    pltpu.sync_copy(edges_hbm.at[c, s], idx_v)
    pltpu.sync_copy(ones_hbm, ones_v)
    plsc.subcore_barrier()
    pend = [None] * 8
    for k in range(CHN):
        b = k % 8
        if pend[b] is not None:
            pend[b].wait()
        pend[b] = pltpu.async_copy(ones_v, acc_sh.at[idx_v.at[k]], sem,
                                   add=True)
    for b in range(8):
        if pend[b] is not None:
            pend[b].wait()
    plsc.subcore_barrier()
    pltpu.sync_copy(acc_sh.at[pl.ds(s * ROWS_PS, ROWS_PS)],
                    out_hbm.at[c, pl.ds(s * ROWS_PS, ROWS_PS)])


def _agg_body(z_hbm, src_hbm, dst_hbm, zeros_hbm, out_hbm, src_v, dst_v,
              bufs, acc_sh, gsems, ssems, zsem):
    c = lax.axis_index("c")
    s = lax.axis_index("s")
    zc = z_hbm.at[c]
    pltpu.sync_copy(src_hbm.at[s], src_v)
    pltpu.sync_copy(dst_hbm.at[s], dst_v)

    gpend = [None] * NB
    spend = [None] * NB
    for k in range(NB - 1):
        gpend[k] = pltpu.async_copy(zc.at[src_v.at[k]], bufs[k], gsems[k])
    zpend = pltpu.async_copy(zeros_hbm,
                             acc_sh.at[pl.ds(s * ROWS_PS, ROWS_PS)], zsem)
    zpend.wait()
    plsc.subcore_barrier()

    for k in range(CHN):
        kn = k + NB - 1
        if kn < CHN:
            bn = kn % NB
            if spend[bn] is not None:
                spend[bn].wait()
            gpend[bn] = pltpu.async_copy(zc.at[src_v.at[kn]], bufs[bn],
                                         gsems[bn])
        b = k % NB
        gpend[b].wait()
        spend[b] = pltpu.async_copy(bufs[b], acc_sh.at[dst_v.at[k]],
                                    ssems[b], add=True)
    for b in range(NB):
        if spend[b] is not None:
            spend[b].wait()
    plsc.subcore_barrier()
    pltpu.sync_copy(acc_sh.at[pl.ds(s * ROWS_PS, ROWS_PS)],
                    out_hbm.at[pl.ds(s * ROWS_PS, ROWS_PS),
                               pl.ds(c * DH, DH)])


_sc_cache = {}


def _sc_kernels():
    if not _sc_cache:
        mesh = plsc.VectorSubcoreMesh(core_axis_name="c",
                                      subcore_axis_name="s",
                                      num_cores=NC, num_subcores=NS)
        params = pltpu.CompilerParams(use_tc_tiling_on_sc=False)
        _sc_cache["deg"] = pl.kernel(
            _deg_body,
            out_type=jax.ShapeDtypeStruct((NC, NPAD, 16), jnp.float32),
            mesh=mesh,
            scratch_types=[
                pltpu.VMEM((CHN, EC), jnp.int32),
                pltpu.VMEM((EC, 16), jnp.float32),
                pltpu.VMEM_SHARED((NPAD, 16), jnp.float32),
                pltpu.SemaphoreType.DMA,
            ],
            compiler_params=params,
        )
        _sc_cache["agg"] = pl.kernel(
            _agg_body,
            out_type=jax.ShapeDtypeStruct((NPAD, D), jnp.float32),
            mesh=mesh,
            scratch_types=[
                pltpu.VMEM((CHN, EC), jnp.int32),
                pltpu.VMEM((CHN, EC), jnp.int32),
                [pltpu.VMEM((EC, DH), jnp.float32) for _ in range(NB)],
                pltpu.VMEM_SHARED((NPAD, DH), jnp.float32),
                [pltpu.SemaphoreType.DMA for _ in range(NB)],
                [pltpu.SemaphoreType.DMA for _ in range(NB)],
                pltpu.SemaphoreType.DMA,
            ],
            compiler_params=params,
        )
    return _sc_cache["deg"], _sc_cache["agg"]


def _norms(deg_ref):
    ns = lax.rsqrt(jnp.maximum(deg_ref[0, :, 0:1], 1.0))
    nd = lax.rsqrt(jnp.maximum(deg_ref[1, :, 0:1], 1.0))
    return ns, nd


def _split(z):
    return jnp.stack([z[:, :DH], z[:, DH:]])


def _mm_first_body(f_ref, deg_ref, w_ref, o_ref):
    ns, _ = _norms(deg_ref)
    o_ref[...] = _split(jnp.dot(f_ref[...] * ns, w_ref[...],
                                preferred_element_type=jnp.float32))


def _mm_mid_body(p_ref, deg_ref, b_ref, w_ref, o_ref):
    ns, nd = _norms(deg_ref)
    h = jnp.maximum(p_ref[...] * nd + b_ref[...], 0.0)
    o_ref[...] = _split(jnp.dot(h * ns, w_ref[...],
                                preferred_element_type=jnp.float32))


def _mm_last_body(p_ref, deg_ref, b_ref, o_ref):
    _, nd = _norms(deg_ref)
    o_ref[...] = p_ref[...] * nd + b_ref[...]


_zfull = jax.ShapeDtypeStruct((NPAD, D), jnp.float32)
_zsplit = jax.ShapeDtypeStruct((NC, NPAD, DH), jnp.float32)
_row_spec = pl.BlockSpec((BR, D), lambda i: (i, 0))
_split_spec = pl.BlockSpec((NC, BR, DH), lambda i: (0, i, 0))
_deg_spec = pl.BlockSpec((2, BR, 16), lambda i: (0, i, 0))
_w_spec = pl.BlockSpec((D, D), lambda i: (0, 0))
_b_spec = pl.BlockSpec((1, D), lambda i: (0, 0))

_mm_first = pl.pallas_call(
    _mm_first_body, grid=(GB,),
    in_specs=[_row_spec, _deg_spec, _w_spec],
    out_specs=_split_spec, out_shape=_zsplit)
_mm_mid = pl.pallas_call(
    _mm_mid_body, grid=(GB,),
    in_specs=[_row_spec, _deg_spec, _b_spec, _w_spec],
    out_specs=_split_spec, out_shape=_zsplit)
_mm_last = pl.pallas_call(
    _mm_last_body, grid=(GB,),
    in_specs=[_row_spec, _deg_spec, _b_spec],
    out_specs=_row_spec, out_shape=_zfull)


def kernel(features, edge_index, W1, b1, W2, b2, W3, b3, W4, b4):
    pad = jnp.full((EPAD - E,), DUMMY, jnp.int32)
    srcp = jnp.concatenate([edge_index[0], pad])
    dstp = jnp.concatenate([edge_index[1], pad])
    srcr = srcp.reshape(NS, CHN, EC)
    dstr = dstp.reshape(NS, CHN, EC)
    edges_deg = jnp.stack([srcr, dstr])
    fpad = jnp.concatenate(
        [features, jnp.zeros((NPAD - N, D), jnp.float32)])

    ones16 = jnp.ones((EC, 16), jnp.float32)
    zeros16 = jnp.zeros((ROWS_PS, 16), jnp.float32)
    zerosD = jnp.zeros((ROWS_PS, DH), jnp.float32)

    deg_kernel, agg_kernel = _sc_kernels()
    deg = deg_kernel(edges_deg, ones16, zeros16)
    z = _mm_first(fpad, deg, W1)
    for (W, b) in ((W2, b1), (W3, b2), (W4, b3)):
        part = agg_kernel(z, srcr, dstr, zerosD)
        z = _mm_mid(part, deg, b.reshape(1, D), W)
    part = agg_kernel(z, srcr, dstr, zerosD)
    out = _mm_last(part, deg, b4.reshape(1, D))
    return out[:N]
